# baseline (speedup 1.0000x reference)
import functools

import jax
import jax.numpy as jnp
from jax import lax
from jax.experimental import pallas as pl
from jax.experimental.pallas import tpu as pltpu
from jax.experimental.pallas import tpu_sc as plsc

NC = 2
NS = 16
NW = NC * NS
CHUNK = 128


def _cdiv(a, b):
    return (a + b - 1) // b


def _gate_body(emb_ref, wg_ref, bg_ref, out_ref):
    x = emb_ref[...]
    out_ref[...] = x * jax.nn.sigmoid(x @ wg_ref[...] + bg_ref[...])


def _gate(emb, Wg, bg2, bm):
    N, D = emb.shape
    return pl.pallas_call(
        _gate_body,
        grid=(N // bm,),
        in_specs=[
            pl.BlockSpec((bm, D), lambda i: (i, 0)),
            pl.BlockSpec((D, D), lambda i: (0, 0)),
            pl.BlockSpec((1, D), lambda i: (0, 0)),
        ],
        out_specs=pl.BlockSpec((bm, D), lambda i: (i, 0)),
        out_shape=jax.ShapeDtypeStruct((N, D), jnp.float32),
    )(emb, Wg, bg2)


def _combine_body(relu, p_ref, d_ref, w_ref, b_ref, out_ref):
    p = p_ref[...]
    d = d_ref[...]
    s = p[0] + p[1]
    deg = d[0] + d[1]
    inv = 1.0 / jnp.maximum(deg, 1.0)
    y = (s * inv) @ w_ref[...] + b_ref[...]
    if relu:
        y = jnp.maximum(y, 0.0)
    out_ref[...] = y


def _combine(partials, degs, W, b2, relu, N, bm):
    D = W.shape[0]
    body = functools.partial(_combine_body, relu)
    return pl.pallas_call(
        body,
        grid=(N // bm,),
        in_specs=[
            pl.BlockSpec((NC, bm, D), lambda i: (0, i, 0)),
            pl.BlockSpec((NC, bm, D), lambda i: (0, i, 0)),
            pl.BlockSpec((D, D), lambda i: (0, 0)),
            pl.BlockSpec((1, D), lambda i: (0, 0)),
        ],
        out_specs=pl.BlockSpec((bm, D), lambda i: (i, 0)),
        out_shape=jax.ShapeDtypeStruct((N, D), jnp.float32),
    )(partials, degs, W, b2)


NBUF = 4
CSEG = 64
NPHASE = 4


def _make_segsum(N, D, cpw, Np, rpw):
    mesh = plsc.VectorSubcoreMesh(core_axis_name="c", subcore_axis_name="s")
    S = cpw // NPHASE

    @functools.partial(
        pl.kernel,
        out_type=jax.ShapeDtypeStruct((NC, Np, D), jnp.float32),
        mesh=mesh,
        scratch_types=[
            pltpu.VMEM((S, CSEG), jnp.int32),
            pltpu.VMEM((S, CSEG), jnp.int32),
            pltpu.VMEM((NBUF, CSEG, D), jnp.float32),
            pltpu.VMEM_SHARED((Np, D), jnp.float32),
            pltpu.SemaphoreType.DMA((NBUF,)),
            pltpu.SemaphoreType.DMA((NBUF,)),
        ],
    )
    def seg(table_hbm, src_hbm, dst_hbm, z_hbm, out_hbm,
            srcbuf, dstbuf, rowbufs, acc, semg, sems):
        core = lax.axis_index("c")
        sub = lax.axis_index("s")
        wid = sub * NC + core

        def gather(j, b):
            return pltpu.make_async_copy(
                table_hbm.at[srcbuf.at[j]], rowbufs.at[b], semg.at[b])

        def scatter(j, b):
            return pltpu.make_async_copy(
                rowbufs.at[b], acc.at[dstbuf.at[j]], sems.at[b])

        pltpu.sync_copy(z_hbm, acc.at[pl.ds(sub * rpw, rpw)])
        plsc.subcore_barrier()

        for p in range(NPHASE):
            pltpu.sync_copy(src_hbm.at[wid, pl.ds(p * S, S)], srcbuf)
            pltpu.sync_copy(dst_hbm.at[wid, pl.ds(p * S, S)], dstbuf)
            gather(0, 0).start()
            gather(1, 1).start()
            for j in (0, 1):
                gather(j + 2, j + 2).start()
                gather(j, j).wait()
                scatter(j, j).start(add=True)

            @pl.loop(0, (S - 4) // 4)
            def _(t):
                base = t * 4
                for b in range(4):
                    sl = (b + 2) % 4
                    scatter(base + b, b).wait()
                    gather(base + 4 + b, b).start()
                    gather(base + 2 + b, sl).wait()
                    scatter(base + 2 + b, sl).start(add=True)

            for j in (S - 2, S - 1):
                scatter(j - 2, (j - 2) % 4).wait()
                gather(j, j % 4).wait()
                scatter(j, j % 4).start(add=True)
            for j in (S - 2, S - 1):
                scatter(j, j % 4).wait()

        plsc.subcore_barrier()
        pltpu.sync_copy(acc.at[pl.ds(sub * rpw, rpw)],
                        out_hbm.at[core, pl.ds(sub * rpw, rpw)])

    return seg


def _make_degree(cpw, Np, rpw):
    mesh = plsc.VectorSubcoreMesh(core_axis_name="c", subcore_axis_name="s")

    @functools.partial(
        pl.kernel,
        out_type=jax.ShapeDtypeStruct((NC, Np, CHUNK), jnp.float32),
        mesh=mesh,
        scratch_types=[
            pltpu.VMEM((cpw, CHUNK), jnp.int32),
            pltpu.VMEM((CHUNK, CHUNK), jnp.float32),
            pltpu.VMEM_SHARED((Np, CHUNK), jnp.float32),
            pltpu.SemaphoreType.DMA,
        ],
    )
    def degk(dst_hbm, ones_hbm, z_hbm, out_hbm, dstbuf, onesbuf, acc, semd):
        core = lax.axis_index("c")
        sub = lax.axis_index("s")
        wid = sub * NC + core
        pltpu.sync_copy(z_hbm, acc.at[pl.ds(sub * rpw, rpw)])
        pltpu.sync_copy(dst_hbm.at[wid], dstbuf)
        pltpu.sync_copy(ones_hbm, onesbuf)
        plsc.subcore_barrier()

        def scat(j):
            return pltpu.make_async_copy(onesbuf, acc.at[dstbuf.at[j]], semd)

        @pl.loop(0, cpw // 8)
        def _(t):
            for b in range(8):
                scat(t * 8 + b).start(add=True)
            for b in range(8):
                scat(t * 8 + b).wait()

        plsc.subcore_barrier()
        pltpu.sync_copy(acc.at[pl.ds(sub * rpw, rpw)],
                        out_hbm.at[core, pl.ds(sub * rpw, rpw)])

    return degk


def _make_lookup(D, lpw, BLp):
    mesh = plsc.VectorSubcoreMesh(core_axis_name="c", subcore_axis_name="s")

    @functools.partial(
        pl.kernel,
        out_type=jax.ShapeDtypeStruct((BLp, D), jnp.float32),
        mesh=mesh,
        scratch_types=[
            pltpu.VMEM((lpw, CHUNK), jnp.int32),
            pltpu.VMEM((NBUF, CHUNK, D), jnp.float32),
            pltpu.SemaphoreType.DMA((NBUF,)),
            pltpu.SemaphoreType.DMA((NBUF,)),
        ],
    )
    def lk(table_hbm, idx_hbm, out_hbm, idxbuf, rowbufs, semg, semw):
        core = lax.axis_index("c")
        sub = lax.axis_index("s")
        wid = sub * NC + core

        def gather(j, b):
            return pltpu.make_async_copy(
                table_hbm.at[idxbuf.at[j]], rowbufs.at[b], semg.at[b])

        def wb(j, b):
            return pltpu.make_async_copy(
                rowbufs.at[b],
                out_hbm.at[pl.ds((wid * lpw + j) * CHUNK, CHUNK)],
                semw.at[b])

        pltpu.sync_copy(idx_hbm.at[wid], idxbuf)
        S = lpw
        gather(0, 0).start()
        gather(1, 1).start()
        for j in (0, 1):
            gather(j + 2, j + 2).start()
            gather(j, j).wait()
            wb(j, j).start()

        @pl.loop(0, (S - 4) // 4)
        def _(t):
            base = t * 4
            for b in range(4):
                sl = (b + 2) % 4
                wb(base + b, b).wait()
                gather(base + 4 + b, b).start()
                gather(base + 2 + b, sl).wait()
                wb(base + 2 + b, sl).start()

        for j in (S - 2, S - 1):
            wb(j - 2, (j - 2) % 4).wait()
            gather(j, j % 4).wait()
            wb(j, j % 4).start()
        for j in (S - 2, S - 1):
            wb(j, j % 4).wait()

    return lk


def kernel(inputs, input_timestamp, input_id, epoch, static_graph,
           emb, Wg, bg, W1, b1, W2, b2):
    N, D = emb.shape
    E = static_graph.shape[1]
    B, L = inputs.shape
    bm = 1000 if N % 1000 == 0 else 8

    Np = _cdiv(N + 1, 8 * NS) * 8 * NS

    cpw = _cdiv(E, NW * CSEG * 4 * NPHASE) * 4 * NPHASE
    E_pad = cpw * NW * CSEG
    src = static_graph[0].astype(jnp.int32)
    dst = static_graph[1].astype(jnp.int32)
    pad_src = jnp.arange(E_pad - E, dtype=jnp.int32) % N
    pad_dst = N + jnp.arange(E_pad - E, dtype=jnp.int32) % (Np - N)
    src_pf = jnp.concatenate([src, pad_src])
    dst_pf = jnp.concatenate([dst, pad_dst])
    src_p = src_pf.reshape(NW, cpw, CSEG)
    dst_p = dst_pf.reshape(NW, cpw, CSEG)
    cpw_d = E_pad // (NW * CHUNK)
    dst_pd = dst_pf.reshape(NW, cpw_d, CHUNK)
    rpw = Np // NS
    zer = jnp.zeros((rpw, D), jnp.float32)
    ones = jnp.ones((CHUNK, CHUNK), jnp.float32)

    bg2 = bg.reshape(1, D)
    b12 = b1.reshape(1, D)
    b22 = b2.reshape(1, D)

    segsum = _make_segsum(N, D, cpw, Np, rpw)

    degs = _make_degree(cpw_d, Np, rpw)(dst_pd, ones, zer)
    h = _gate(emb, Wg, bg2, bm)
    part1 = segsum(h, src_p, dst_p, zer)
    h1 = _combine(part1, degs, W1, b12, True, N, bm)
    part2 = segsum(h1, src_p, dst_p, zer)
    all_st = _combine(part2, degs, W2, b22, False, N, bm)

    BL = B * L
    lpw = _cdiv(BL, NW * CHUNK * 4) * 4
    BLp = lpw * NW * CHUNK
    pad_idx = jnp.arange(BLp - BL, dtype=jnp.int32) % N
    idx = inputs.reshape(-1).astype(jnp.int32)
    idx = jnp.concatenate([idx, pad_idx]).reshape(NW, lpw, CHUNK)
    rows = _make_lookup(D, lpw, BLp)(all_st, idx)
    user_st_seq_rep = rows[:BL].reshape(B, L, D)
    return (user_st_seq_rep, all_st)


# speedup vs baseline: 1.1465x; 1.1465x over previous
import functools

import jax
import jax.numpy as jnp
from jax import lax
from jax.experimental import pallas as pl
from jax.experimental.pallas import tpu as pltpu
from jax.experimental.pallas import tpu_sc as plsc

NC = 2
NS = 16
NW = NC * NS
CHUNK = 128


def _cdiv(a, b):
    return (a + b - 1) // b


def _gate_body(emb_ref, wg_ref, bg_ref, out_ref):
    x = emb_ref[...]
    out_ref[...] = x * jax.nn.sigmoid(x @ wg_ref[...] + bg_ref[...])


def _gate(emb, Wg, bg2, bm):
    N, D = emb.shape
    return pl.pallas_call(
        _gate_body,
        grid=(N // bm,),
        in_specs=[
            pl.BlockSpec((bm, D), lambda i: (i, 0)),
            pl.BlockSpec((D, D), lambda i: (0, 0)),
            pl.BlockSpec((1, D), lambda i: (0, 0)),
        ],
        out_specs=pl.BlockSpec((bm, D), lambda i: (i, 0)),
        out_shape=jax.ShapeDtypeStruct((N, D), jnp.float32),
    )(emb, Wg, bg2)


def _combine_body(relu, p_ref, d_ref, w_ref, b_ref, out_ref):
    p = p_ref[...]
    d = d_ref[...]
    s = p[0] + p[1]
    deg = d[0] + d[1]
    inv = 1.0 / jnp.maximum(deg, 1.0)
    y = (s * inv) @ w_ref[...] + b_ref[...]
    if relu:
        y = jnp.maximum(y, 0.0)
    out_ref[...] = y


def _combine(partials, degs, W, b2, relu, N, bm):
    D = W.shape[0]
    body = functools.partial(_combine_body, relu)
    return pl.pallas_call(
        body,
        grid=(N // bm,),
        in_specs=[
            pl.BlockSpec((NC, bm, D), lambda i: (0, i, 0)),
            pl.BlockSpec((NC, bm, D), lambda i: (0, i, 0)),
            pl.BlockSpec((D, D), lambda i: (0, 0)),
            pl.BlockSpec((1, D), lambda i: (0, 0)),
        ],
        out_specs=pl.BlockSpec((bm, D), lambda i: (i, 0)),
        out_shape=jax.ShapeDtypeStruct((N, D), jnp.float32),
    )(partials, degs, W, b2)


NBUF = 4
CSEG = 64
NPHASE = 4


def _make_segsum(N, D, cpw, Np, rpw):
    mesh = plsc.VectorSubcoreMesh(core_axis_name="c", subcore_axis_name="s")
    S = cpw // NPHASE

    @functools.partial(
        pl.kernel,
        out_type=jax.ShapeDtypeStruct((NC, Np, D), jnp.float32),
        mesh=mesh,
        scratch_types=[
            pltpu.VMEM((S, CSEG), jnp.int32),
            pltpu.VMEM((S, CSEG), jnp.int32),
            pltpu.VMEM((NBUF, CSEG, D), jnp.float32),
            pltpu.VMEM_SHARED((Np, D), jnp.float32),
            pltpu.SemaphoreType.DMA((NBUF,)),
            pltpu.SemaphoreType.DMA((NBUF,)),
        ],
    )
    def seg(table_hbm, src_hbm, dst_hbm, z_hbm, out_hbm,
            srcbuf, dstbuf, rowbufs, acc, semg, sems):
        core = lax.axis_index("c")
        sub = lax.axis_index("s")
        wid = sub * NC + core

        def gather(j, b):
            return pltpu.make_async_copy(
                table_hbm.at[srcbuf.at[j]], rowbufs.at[b], semg.at[b])

        def scatter(j, b):
            return pltpu.make_async_copy(
                rowbufs.at[b], acc.at[dstbuf.at[j]], sems.at[b])

        pltpu.sync_copy(z_hbm, acc.at[pl.ds(sub * rpw, rpw)])
        plsc.subcore_barrier()

        for p in range(NPHASE):
            pltpu.sync_copy(src_hbm.at[wid, pl.ds(p * S, S)], srcbuf)
            pltpu.sync_copy(dst_hbm.at[wid, pl.ds(p * S, S)], dstbuf)
            gather(0, 0).start()
            gather(1, 1).start()
            for j in (0, 1):
                gather(j + 2, j + 2).start()
                gather(j, j).wait()
                scatter(j, j).start(add=True)

            @pl.loop(0, (S - 4) // 4)
            def _(t):
                base = t * 4
                for b in range(4):
                    sl = (b + 2) % 4
                    scatter(base + b, b).wait()
                    gather(base + 4 + b, b).start()
                    gather(base + 2 + b, sl).wait()
                    scatter(base + 2 + b, sl).start(add=True)

            for j in (S - 2, S - 1):
                scatter(j - 2, (j - 2) % 4).wait()
                gather(j, j % 4).wait()
                scatter(j, j % 4).start(add=True)
            for j in (S - 2, S - 1):
                scatter(j, j % 4).wait()

        plsc.subcore_barrier()
        pltpu.sync_copy(acc.at[pl.ds(sub * rpw, rpw)],
                        out_hbm.at[core, pl.ds(sub * rpw, rpw)])

    return seg


def _make_degree(cpw, Np, rpw):
    mesh = plsc.VectorSubcoreMesh(core_axis_name="c", subcore_axis_name="s")

    @functools.partial(
        pl.kernel,
        out_type=jax.ShapeDtypeStruct((NC, Np, CHUNK), jnp.float32),
        mesh=mesh,
        scratch_types=[
            pltpu.VMEM((cpw, CHUNK), jnp.int32),
            pltpu.VMEM((CHUNK, CHUNK), jnp.float32),
            pltpu.VMEM_SHARED((Np, CHUNK), jnp.float32),
            pltpu.SemaphoreType.DMA,
        ],
    )
    def degk(dst_hbm, ones_hbm, z_hbm, out_hbm, dstbuf, onesbuf, acc, semd):
        core = lax.axis_index("c")
        sub = lax.axis_index("s")
        wid = sub * NC + core
        pltpu.sync_copy(z_hbm, acc.at[pl.ds(sub * rpw, rpw)])
        pltpu.sync_copy(dst_hbm.at[wid], dstbuf)
        pltpu.sync_copy(ones_hbm, onesbuf)
        plsc.subcore_barrier()

        def scat(j):
            return pltpu.make_async_copy(onesbuf, acc.at[dstbuf.at[j]], semd)

        @pl.loop(0, cpw // 8)
        def _(t):
            for b in range(8):
                scat(t * 8 + b).start(add=True)
            for b in range(8):
                scat(t * 8 + b).wait()

        plsc.subcore_barrier()
        pltpu.sync_copy(acc.at[pl.ds(sub * rpw, rpw)],
                        out_hbm.at[core, pl.ds(sub * rpw, rpw)])

    return degk


def _make_lookup(D, lpw, BLp):
    mesh = plsc.VectorSubcoreMesh(core_axis_name="c", subcore_axis_name="s")

    @functools.partial(
        pl.kernel,
        out_type=jax.ShapeDtypeStruct((BLp, D), jnp.float32),
        mesh=mesh,
        scratch_types=[
            pltpu.VMEM((lpw, CHUNK), jnp.int32),
            pltpu.VMEM((NBUF, CHUNK, D), jnp.float32),
            pltpu.SemaphoreType.DMA((NBUF,)),
            pltpu.SemaphoreType.DMA((NBUF,)),
        ],
    )
    def lk(table_hbm, idx_hbm, out_hbm, idxbuf, rowbufs, semg, semw):
        core = lax.axis_index("c")
        sub = lax.axis_index("s")
        wid = sub * NC + core

        def gather(j, b):
            return pltpu.make_async_copy(
                table_hbm.at[idxbuf.at[j]], rowbufs.at[b], semg.at[b])

        def wb(j, b):
            return pltpu.make_async_copy(
                rowbufs.at[b],
                out_hbm.at[pl.ds((wid * lpw + j) * CHUNK, CHUNK)],
                semw.at[b])

        pltpu.sync_copy(idx_hbm.at[wid], idxbuf)
        S = lpw
        for j in range(min(2, S)):
            gather(j, j % NBUF).start()
        for j in range(S):
            if j >= 2:
                wb(j - 2, (j - 2) % NBUF).wait()
            if j + 2 < S:
                gather(j + 2, (j + 2) % NBUF).start()
            gather(j, j % NBUF).wait()
            wb(j, j % NBUF).start()
        for j in range(max(0, S - 2), S):
            wb(j, j % NBUF).wait()

    return lk


def kernel(inputs, input_timestamp, input_id, epoch, static_graph,
           emb, Wg, bg, W1, b1, W2, b2):
    N, D = emb.shape
    E = static_graph.shape[1]
    B, L = inputs.shape
    bm = 1000 if N % 1000 == 0 else 8

    Np = _cdiv(N + 1, 8 * NS) * 8 * NS

    cpw = _cdiv(E, NW * CSEG * 4 * NPHASE) * 4 * NPHASE
    E_pad = cpw * NW * CSEG
    src = static_graph[0].astype(jnp.int32)
    dst = static_graph[1].astype(jnp.int32)
    pad_src = jnp.arange(E_pad - E, dtype=jnp.int32) % N
    pad_dst = N + jnp.arange(E_pad - E, dtype=jnp.int32) % (Np - N)
    src_pf = jnp.concatenate([src, pad_src])
    dst_pf = jnp.concatenate([dst, pad_dst])
    src_p = src_pf.reshape(NW, cpw, CSEG)
    dst_p = dst_pf.reshape(NW, cpw, CSEG)
    cpw_d = E_pad // (NW * CHUNK)
    dst_pd = dst_pf.reshape(NW, cpw_d, CHUNK)
    rpw = Np // NS
    zer = jnp.zeros((rpw, D), jnp.float32)
    ones = jnp.ones((CHUNK, CHUNK), jnp.float32)

    bg2 = bg.reshape(1, D)
    b12 = b1.reshape(1, D)
    b22 = b2.reshape(1, D)

    segsum = _make_segsum(N, D, cpw, Np, rpw)

    degs = _make_degree(cpw_d, Np, rpw)(dst_pd, ones, zer)
    h = _gate(emb, Wg, bg2, bm)
    part1 = segsum(h, src_p, dst_p, zer)
    h1 = _combine(part1, degs, W1, b12, True, N, bm)
    part2 = segsum(h1, src_p, dst_p, zer)
    all_st = _combine(part2, degs, W2, b22, False, N, bm)

    BL = B * L
    lpw = _cdiv(BL, NW * CHUNK)
    BLp = lpw * NW * CHUNK
    pad_idx = jnp.arange(BLp - BL, dtype=jnp.int32) % N
    idx = inputs.reshape(-1).astype(jnp.int32)
    idx = jnp.concatenate([idx, pad_idx]).reshape(NW, lpw, CHUNK)
    rows = _make_lookup(D, lpw, BLp)(all_st, idx)
    user_st_seq_rep = rows[:BL].reshape(B, L, D)
    return (user_st_seq_rep, all_st)
